# Initial kernel scaffold; baseline (speedup 1.0000x reference)
import jax
import jax.numpy as jnp
from jax.experimental import pallas as pl


def kernel(x, edge_index, W_rel0, b_rel0, W_root0, W_rel1, b_rel1, W_root1, W_logit, b_logit, W_val, b_val):
    raise NotImplementedError("write your pallas kernel here")


# speedup vs baseline: 3.3651x; 3.3651x over previous
import functools

import jax
import jax.numpy as jnp
from jax import lax
from jax.experimental import pallas as pl
from jax.experimental.pallas import tpu as pltpu
from jax.experimental.pallas import tpu_sc as plsc

N = 10000
E = 320000
D = 128
H = 256
O = 1024
NUM_OUT = 18

NSUB = 16
CHUNK = 128
NPC = 160
EPAD = NSUB * CHUNK * NPC

RHALF = 5120
NROWS0 = 5248
ZSTR0 = NROWS0 // NSUB
CSTR0 = RHALF // NSUB


def _sc_seg_sum_l1(table_flat, srcs3d, dstA2d, dstB2d):
    mesh = plsc.VectorSubcoreMesh(core_axis_name="c", subcore_axis_name="s")
    zeros = jnp.zeros((NROWS0, 128), jnp.float32)

    @functools.partial(
        pl.kernel,
        mesh=mesh,
        out_type=jax.ShapeDtypeStruct((2, 2 * RHALF, 128), jnp.float32),
        scratch_types=[
            pltpu.VMEM((NPC, CHUNK), jnp.int32),
            pltpu.VMEM((NPC, CHUNK), jnp.int32),
            pltpu.VMEM((NPC, CHUNK), jnp.int32),
            pltpu.VMEM((CHUNK, 128), jnp.float32),
            pltpu.VMEM_SHARED((NROWS0, 128), jnp.float32),
            pltpu.SemaphoreType.DMA,
        ],
    )
    def k(table_hbm, srcs_hbm, dstA_hbm, dstB_hbm, zero_hbm, out_hbm,
          src_v, dstA_v, dstB_v, rows_v, agg_sh, sem):
        c = lax.axis_index("c")
        s = lax.axis_index("s")
        pltpu.sync_copy(srcs_hbm.at[c, pl.ds(s * NPC, NPC)], src_v)
        pltpu.sync_copy(dstA_hbm.at[pl.ds(s * NPC, NPC)], dstA_v)
        pltpu.sync_copy(dstB_hbm.at[pl.ds(s * NPC, NPC)], dstB_v)

        for dst_v, out_base in ((dstA_v, 0), (dstB_v, RHALF)):
            pltpu.sync_copy(zero_hbm.at[pl.ds(s * ZSTR0, ZSTR0)],
                            agg_sh.at[pl.ds(s * ZSTR0, ZSTR0)])
            plsc.subcore_barrier()

            @pl.loop(0, NPC)
            def _(i):
                pltpu.async_copy(table_hbm.at[src_v.at[i]], rows_v, sem).wait()
                pltpu.sync_copy(rows_v, agg_sh.at[dst_v.at[i]], add=True)

            plsc.subcore_barrier()
            pltpu.sync_copy(
                agg_sh.at[pl.ds(s * CSTR0, CSTR0)],
                out_hbm.at[c, pl.ds(out_base + s * CSTR0, CSTR0)])
            plsc.subcore_barrier()

    return k(table_flat, srcs3d, dstA2d, dstB2d, zeros)


def _sc_seg_sum_l0(table, src2d, dstA2d, dstB2d):
    mesh = plsc.VectorSubcoreMesh(core_axis_name="c", subcore_axis_name="s")
    zeros = jnp.zeros((NROWS0, 128), jnp.float32)
    npc = NPC // 2

    @functools.partial(
        pl.kernel,
        mesh=mesh,
        out_type=jax.ShapeDtypeStruct((2, 2 * RHALF, 128), jnp.float32),
        scratch_types=[
            pltpu.VMEM((npc, CHUNK), jnp.int32),
            pltpu.VMEM((npc, CHUNK), jnp.int32),
            pltpu.VMEM((npc, CHUNK), jnp.int32),
            pltpu.VMEM((CHUNK, 128), jnp.float32),
            pltpu.VMEM_SHARED((NROWS0, 128), jnp.float32),
            pltpu.SemaphoreType.DMA,
        ],
    )
    def k(table_hbm, src_hbm, dstA_hbm, dstB_hbm, zero_hbm, out_hbm,
          src_v, dstA_v, dstB_v, rows_v, agg_sh, sem):
        c = lax.axis_index("c")
        s = lax.axis_index("s")
        base = (c * NSUB + s) * npc
        pltpu.sync_copy(src_hbm.at[pl.ds(base, npc)], src_v)
        pltpu.sync_copy(dstA_hbm.at[pl.ds(base, npc)], dstA_v)
        pltpu.sync_copy(dstB_hbm.at[pl.ds(base, npc)], dstB_v)

        for dst_v, out_base in ((dstA_v, 0), (dstB_v, RHALF)):
            pltpu.sync_copy(zero_hbm.at[pl.ds(s * ZSTR0, ZSTR0)],
                            agg_sh.at[pl.ds(s * ZSTR0, ZSTR0)])
            plsc.subcore_barrier()

            @pl.loop(0, npc)
            def _(i):
                pltpu.async_copy(table_hbm.at[src_v.at[i]], rows_v, sem).wait()
                pltpu.sync_copy(rows_v, agg_sh.at[dst_v.at[i]], add=True)

            plsc.subcore_barrier()
            pltpu.sync_copy(
                agg_sh.at[pl.ds(s * CSTR0, CSTR0)],
                out_hbm.at[c, pl.ds(out_base + s * CSTR0, CSTR0)])
            plsc.subcore_barrier()

    return k(table, src2d, dstA2d, dstB2d, zeros)


BN = 1000


def _tc1_body(x_ref, a_ref, wrel_ref, wroot_ref, b_ref, h_ref):
    agg = a_ref[0] + a_ref[1]
    h = (jnp.dot(agg, wrel_ref[...], preferred_element_type=jnp.float32)
         + jnp.dot(x_ref[...], wroot_ref[...], preferred_element_type=jnp.float32)
         + b_ref[...])
    h = jnp.maximum(h, 0.0)
    h_ref[0] = h[:, 0:128]
    h_ref[1] = h[:, 128:256]


def _tc1(x, agg_partials, W_rel0, W_root0, b_rel0):
    return pl.pallas_call(
        _tc1_body,
        grid=(N // BN,),
        in_specs=[
            pl.BlockSpec((BN, D), lambda i: (i, 0)),
            pl.BlockSpec((2, BN, 128), lambda i: (0, i, 0)),
            pl.BlockSpec((D, H), lambda i: (0, 0)),
            pl.BlockSpec((D, H), lambda i: (0, 0)),
            pl.BlockSpec((1, H), lambda i: (0, 0)),
        ],
        out_specs=pl.BlockSpec((2, BN, 128), lambda i: (0, i, 0)),
        out_shape=jax.ShapeDtypeStruct((2, N, 128), jnp.float32),
    )(x, agg_partials, W_rel0, W_root0, b_rel0.reshape(1, H))


def _tc2_body(a_ref, h_ref, wrel_ref, wroot_ref, b_ref, wlv_ref, blv_ref, lv_ref):
    o = (jnp.dot(a_ref[0], wrel_ref[0:128, :], preferred_element_type=jnp.float32)
         + jnp.dot(a_ref[1], wrel_ref[128:256, :], preferred_element_type=jnp.float32)
         + jnp.dot(h_ref[0], wroot_ref[0:128, :], preferred_element_type=jnp.float32)
         + jnp.dot(h_ref[1], wroot_ref[128:256, :], preferred_element_type=jnp.float32)
         + b_ref[...])
    o = jnp.maximum(o, 0.0)
    lv_ref[...] = jnp.dot(o, wlv_ref[...], preferred_element_type=jnp.float32) + blv_ref[...]


def _tc2(agg_parts, h_parts, W_rel1, W_root1, b_rel1, Wlv, blv):
    return pl.pallas_call(
        _tc2_body,
        grid=(N // BN,),
        in_specs=[
            pl.BlockSpec((2, BN, 128), lambda i: (0, i, 0)),
            pl.BlockSpec((2, BN, 128), lambda i: (0, i, 0)),
            pl.BlockSpec((H, O), lambda i: (0, 0)),
            pl.BlockSpec((H, O), lambda i: (0, 0)),
            pl.BlockSpec((1, O), lambda i: (0, 0)),
            pl.BlockSpec((O, 128), lambda i: (0, 0)),
            pl.BlockSpec((1, 128), lambda i: (0, 0)),
        ],
        out_specs=pl.BlockSpec((BN, 128), lambda i: (i, 0)),
        out_shape=jax.ShapeDtypeStruct((N, 128), jnp.float32),
    )(agg_parts, h_parts, W_rel1, W_root1, b_rel1.reshape(1, O), Wlv, blv)


def kernel(x, edge_index, W_rel0, b_rel0, W_root0, W_rel1, b_rel1, W_root1,
           W_logit, b_logit, W_val, b_val):
    src = edge_index[0].astype(jnp.int32)
    dst = edge_index[1].astype(jnp.int32)
    pad = EPAD - E
    ar = jnp.arange(pad, dtype=jnp.int32)
    src_p = jnp.concatenate([src, ar % NSUB]).reshape(-1, CHUNK)

    ie = jnp.arange(EPAD, dtype=jnp.int32) % CHUNK
    dst_full = jnp.concatenate([dst, N + (ar % 240)])
    dstA = jnp.where(dst_full < RHALF, dst_full,
                     RHALF + ie).reshape(-1, CHUNK)
    dstB = jnp.where(dst_full >= RHALF, dst_full - RHALF,
                     (RHALF - 240) + ie).reshape(-1, CHUNK)

    agg0 = _sc_seg_sum_l0(x, src_p, dstA, dstB)
    h_parts = _tc1(x, agg0, W_rel0, W_root0, b_rel0)

    srcs2 = jnp.stack([src_p, src_p + N])
    agg1 = _sc_seg_sum_l1(h_parts.reshape(2 * N, 128), srcs2, dstA, dstB)

    Wlv = jnp.concatenate(
        [W_logit, W_val, jnp.zeros((O, 128 - NUM_OUT - 1), jnp.float32)], axis=1)
    blv = jnp.concatenate(
        [b_logit, b_val, jnp.zeros((128 - NUM_OUT - 1,), jnp.float32)]).reshape(1, 128)
    lv = _tc2(agg1, h_parts, W_rel1, W_root1, b_rel1, Wlv, blv)
    logits = lv[:, :NUM_OUT]
    values = lv[:, NUM_OUT:NUM_OUT + 1]
    return logits, values
